# baseline (speedup 1.0000x reference)
import functools

import jax
import jax.numpy as jnp
from jax import lax
from jax.experimental import pallas as pl
from jax.experimental.pallas import tpu as pltpu
from jax.experimental.pallas import tpu_sc as plsc

N = 10000
E = 320000
NC = 2
NS = 16
NW = NC * NS
K = 48
BL = 12
NB = 18
CH = BL * NB
EPT = CH * K
STRIPE = 640
A_ROWS = NS * STRIPE
DUMMY = A_ROWS - 1
Wm = 128
CNT_ROWS = A_ROWS // 128
GRID = 5
RB = N // GRID


def _make_sc_edge(F, GW, has_cnt):
    mesh = plsc.VectorSubcoreMesh(core_axis_name="c", subcore_axis_name="s",
                                  num_cores=NC, num_subcores=NS)

    out_types = [jax.ShapeDtypeStruct((NC * A_ROWS, Wm), jnp.float32)]
    scratch = [
        pltpu.VMEM_SHARED((A_ROWS, Wm), jnp.float32),
        pltpu.VMEM((BL, K), jnp.int32),
        pltpu.VMEM((BL, K), jnp.int32),
        pltpu.VMEM((BL * K,), jnp.float32),
        pltpu.VMEM((BL, K), jnp.int32),
        pltpu.VMEM((BL, K), jnp.int32),
        pltpu.VMEM((BL * K,), jnp.float32),
        pltpu.VMEM((K, GW), jnp.float32),
        pltpu.VMEM((K, GW), jnp.float32),
        pltpu.VMEM((K, Wm), jnp.float32),
        pltpu.SemaphoreType.DMA,
        pltpu.SemaphoreType.DMA,
        pltpu.SemaphoreType.DMA,
        pltpu.SemaphoreType.DMA,
    ]
    if not has_cnt:
        scratch += [
            pltpu.VMEM((K, Wm), jnp.float32),
            pltpu.SemaphoreType.DMA,
        ]
    if has_cnt:
        out_types.append(jax.ShapeDtypeStruct((NC * CNT_ROWS, 128), jnp.float32))
        scratch += [
            pltpu.VMEM_SHARED((CNT_ROWS, 128), jnp.float32),
            pltpu.VMEM((K,), jnp.int32),
            pltpu.VMEM((K, 128), jnp.float32),
        ]

    @functools.partial(
        pl.kernel,
        out_type=out_types,
        mesh=mesh,
        compiler_params=pltpu.CompilerParams(needs_layout_passes=False),
        scratch_types=scratch,
    )
    def sck(hcat, srcv, dstv, pv, *rest):
        if has_cnt:
            (out, out_cnt, acc, s0_v, d0_v, p0_v, s1_v, d1_v, p1_v,
             rows0, rows1, mbuf0, semb0, semb1, semg0, semg1,
             cacc, ccol_v, ohbuf) = rest
        else:
            (out, acc, s0_v, d0_v, p0_v, s1_v, d1_v, p1_v,
             rows0, rows1, mbuf0, semb0, semb1, semg0, semg1,
             mbuf1, semm) = rest
        mbuf = mbuf0
        blks = ((s0_v, d0_v, p0_v, semb0), (s1_v, d1_v, p1_v, semb1))
        cid = lax.axis_index("c")
        sid = lax.axis_index("s")
        wid = cid * NS + sid
        r0 = sid * STRIPE
        base = wid * NB

        def zrow(r, _):
            for j in range(Wm // 16):
                mbuf[r, pl.ds(j * 16, 16)] = jnp.zeros((16,), jnp.float32)
            return 0
        lax.fori_loop(0, K, zrow, 0)

        def zcp(r, _):
            pltpu.sync_copy(mbuf.at[pl.ds(0, 8)],
                            acc.at[pl.ds(r0 + r * 8, 8)])
            return 0
        lax.fori_loop(0, STRIPE // 8, zcp, 0)
        if has_cnt:
            @pl.when(sid < CNT_ROWS // 8)
            def _():
                pltpu.sync_copy(mbuf.at[pl.ds(0, 8)],
                                cacc.at[pl.ds(sid * 8, 8)])

        lanes = lax.iota(jnp.int32, 16)

        def load_block(b, bufs):
            s_r, d_r, p_r, sem = bufs
            pltpu.async_copy(srcv.at[base + b], s_r, sem)
            pltpu.async_copy(dstv.at[base + b], d_r, sem)
            pltpu.async_copy(pv.at[base + b], p_r, sem)

        def wait_block(b, bufs):
            s_r, d_r, p_r, sem = bufs
            pltpu.make_async_copy(srcv.at[base + b], s_r, sem).wait()
            pltpu.make_async_copy(dstv.at[base + b], d_r, sem).wait()
            pltpu.make_async_copy(pv.at[base + b], p_r, sem).wait()

        def startg(s_r, ci, rbuf, sem):
            pltpu.async_copy(hcat.at[s_r.at[ci]], rbuf, sem)

        def waitg(s_r, ci, rbuf, sem):
            pltpu.make_async_copy(hcat.at[s_r.at[ci]], rbuf, sem).wait()

        def compute(bufs, ci, rows_ref, mb, wait_spec=None):
            s_r, d_r, p_r, sem = bufs
            if has_cnt:
                def gbody(g, _):
                    dv = d_r[ci, pl.ds(g * 16, 16)]
                    ccol_v[pl.ds(g * 16, 16)] = jnp.bitwise_and(dv, 127)
                    return 0
                lax.fori_loop(0, K // 16, gbody, 0)

            @plsc.parallel_loop(0, K, unroll=4)
            def _(e):
                pe = plsc.load_gather(
                    p_r, [jnp.full((16,), ci * K + e, jnp.int32)])
                for j in range(F // 16):
                    g = rows_ref[e, pl.ds(j * 16, 16)]
                    d = rows_ref[e, pl.ds(F + j * 16, 16)]
                    mb[e, pl.ds(j * 16, 16)] = g + pe * d
                if has_cnt:
                    cv = plsc.load_gather(
                        ccol_v, [jnp.full((16,), e, jnp.int32)])
                    for j in range(8):
                        ohbuf[e, pl.ds(j * 16, 16)] = jnp.where(
                            lanes + (j * 16) == cv, 1.0, 0.0)
            if has_cnt:
                pltpu.sync_copy(mb, acc.at[d_r.at[ci]], add=True)
            else:
                wcond, wci, wmb = wait_spec
                if wcond is True:
                    pltpu.make_async_copy(
                        wmb, acc.at[d_r.at[wci]], semm).wait()
                elif wcond is not False:
                    @pl.when(wcond)
                    def _():
                        pltpu.make_async_copy(
                            wmb, acc.at[d_r.at[wci]], semm).wait()
                pltpu.async_copy(mb, acc.at[d_r.at[ci]], semm, add=True)
            if has_cnt:
                def hbody(g, _):
                    dv = d_r[ci, pl.ds(g * 16, 16)]
                    cidx16 = lax.shift_right_logical(dv, 7)
                    ccol_v[pl.ds(g * 16, 16)] = cidx16
                    return 0
                lax.fori_loop(0, K // 16, hbody, 0)
                pltpu.sync_copy(ohbuf, cacc.at[ccol_v], add=True)

        load_block(0, blks[0])
        wait_block(0, blks[0])
        load_block(1, blks[1])
        plsc.subcore_barrier()
        startg(s0_v, 0, rows0, semg0)

        def pair_loop(bufs, b, j, _):
            s_r = bufs[0]
            c0 = 2 * j
            startg(s_r, c0 + 1, rows1, semg1)
            waitg(s_r, c0, rows0, semg0)
            if has_cnt:
                compute(bufs, c0, rows0, mbuf0)
            else:
                compute(bufs, c0, rows0, mbuf0, (j >= 1, c0 - 1, mbuf1))
            startg(s_r, c0 + 2, rows0, semg0)
            waitg(s_r, c0 + 1, rows1, semg1)
            if has_cnt:
                compute(bufs, c0 + 1, rows1, mbuf0)
            else:
                compute(bufs, c0 + 1, rows1, mbuf1, (True, c0, mbuf0))
            return 0

        def block_body(b, bufs, nbufs):
            lax.fori_loop(0, BL // 2 - 1,
                          functools.partial(pair_loop, bufs, b), 0)
            s_r = bufs[0]
            d_r = bufs[1]
            startg(s_r, BL - 1, rows1, semg1)
            waitg(s_r, BL - 2, rows0, semg0)
            if has_cnt:
                compute(bufs, BL - 2, rows0, mbuf0)
            else:
                compute(bufs, BL - 2, rows0, mbuf0, (True, BL - 3, mbuf1))

            @pl.when(b + 1 < NB)
            def _():
                wait_block(b + 1, nbufs)
                startg(nbufs[0], 0, rows0, semg0)

            waitg(s_r, BL - 1, rows1, semg1)
            if has_cnt:
                compute(bufs, BL - 1, rows1, mbuf0)
            else:
                compute(bufs, BL - 1, rows1, mbuf1, (True, BL - 2, mbuf0))
                pltpu.make_async_copy(
                    mbuf1, acc.at[d_r.at[BL - 1]], semm).wait()

            @pl.when(b + 2 < NB)
            def _():
                load_block(b + 2, bufs)

        def big_body(i, _):
            block_body(2 * i, blks[0], blks[1])
            block_body(2 * i + 1, blks[1], blks[0])
            return 0
        lax.fori_loop(0, NB // 2, big_body, 0)

        plsc.subcore_barrier()
        pltpu.sync_copy(acc.at[pl.ds(r0, STRIPE)],
                        out.at[pl.ds(cid * A_ROWS + r0, STRIPE)])
        if has_cnt:
            @pl.when(sid < CNT_ROWS // 8)
            def _():
                pltpu.sync_copy(
                    cacc.at[pl.ds(sid * 8, 8)],
                    out_cnt.at[pl.ds(cid * CNT_ROWS + sid * 8, 8)])

    return sck


_sc_edge_cnt = _make_sc_edge(128, 256, True)
_sc_edge_128 = _make_sc_edge(128, 256, False)
_sc_edge_48 = _make_sc_edge(48, 128, False)


def _mm(x, wg, r):
    n, din = x.shape
    fg = wg.shape[1]
    fo = r.shape[1]

    def body(x_ref, wg_ref, r_ref, h_ref, xr_ref):
        xb = x_ref[...]
        h_ref[...] = jnp.dot(xb, wg_ref[...], preferred_element_type=jnp.float32)
        xr_ref[...] = jnp.dot(xb, r_ref[...], preferred_element_type=jnp.float32)

    return pl.pallas_call(
        body,
        grid=(GRID,),
        in_specs=[
            pl.BlockSpec((RB, din), lambda i: (i, 0)),
            pl.BlockSpec((din, fg), lambda i: (0, 0)),
            pl.BlockSpec((din, fo), lambda i: (0, 0)),
        ],
        out_specs=[
            pl.BlockSpec((RB, fg), lambda i: (i, 0)),
            pl.BlockSpec((RB, fo), lambda i: (i, 0)),
        ],
        out_shape=[
            jax.ShapeDtypeStruct((n, fg), jnp.float32),
            jax.ShapeDtypeStruct((n, fo), jnp.float32),
        ],
    )(x, wg, r)


def _p1_first(part, cnt, xr, b):
    F = xr.shape[1]

    def body(part_ref, cnt_ref, xr_ref, b_ref, y_ref, sums_ref, inv_ref):
        i = pl.program_id(0)
        t = part_ref[0, :, :F] + part_ref[1, :, :F]
        cntv = cnt_ref[0] + cnt_ref[1]
        inv = 1.0 / jnp.maximum(cntv, 1.0)
        inv_ref[...] = inv
        y = t * inv + xr_ref[...] + b_ref[...]
        y_ref[...] = y

        @pl.when(i == 0)
        def _():
            sums_ref[...] = jnp.zeros_like(sums_ref)
        sums_ref[0, :] += jnp.sum(y, 0)
        sums_ref[1, :] += jnp.sum(y * y, 0)

    return pl.pallas_call(
        body,
        grid=(GRID,),
        in_specs=[
            pl.BlockSpec((2, RB, Wm), lambda i: (0, i, 0)),
            pl.BlockSpec((2, RB, 1), lambda i: (0, i, 0)),
            pl.BlockSpec((RB, F), lambda i: (i, 0)),
            pl.BlockSpec((1, F), lambda i: (0, 0)),
        ],
        out_specs=[
            pl.BlockSpec((RB, F), lambda i: (i, 0)),
            pl.BlockSpec((2, F), lambda i: (0, 0)),
            pl.BlockSpec((RB, 1), lambda i: (i, 0)),
        ],
        out_shape=[
            jax.ShapeDtypeStruct((N, F), jnp.float32),
            jax.ShapeDtypeStruct((2, F), jnp.float32),
            jax.ShapeDtypeStruct((N, 1), jnp.float32),
        ],
    )(part, cnt, xr, b.reshape(1, F))


def _p1(part, xr, b, inv):
    F = xr.shape[1]

    def body(part_ref, xr_ref, b_ref, inv_ref, y_ref, sums_ref):
        i = pl.program_id(0)
        t = part_ref[0, :, :F] + part_ref[1, :, :F]
        y = t * inv_ref[...] + xr_ref[...] + b_ref[...]
        y_ref[...] = y

        @pl.when(i == 0)
        def _():
            sums_ref[...] = jnp.zeros_like(sums_ref)
        sums_ref[0, :] += jnp.sum(y, 0)
        sums_ref[1, :] += jnp.sum(y * y, 0)

    return pl.pallas_call(
        body,
        grid=(GRID,),
        in_specs=[
            pl.BlockSpec((2, RB, Wm), lambda i: (0, i, 0)),
            pl.BlockSpec((RB, F), lambda i: (i, 0)),
            pl.BlockSpec((1, F), lambda i: (0, 0)),
            pl.BlockSpec((RB, 1), lambda i: (i, 0)),
        ],
        out_specs=[
            pl.BlockSpec((RB, F), lambda i: (i, 0)),
            pl.BlockSpec((2, F), lambda i: (0, 0)),
        ],
        out_shape=[
            jax.ShapeDtypeStruct((N, F), jnp.float32),
            jax.ShapeDtypeStruct((2, F), jnp.float32),
        ],
    )(part, xr, b.reshape(1, F), inv)


def _p2(y, sums, g, be):
    F = y.shape[1]

    def body(y_ref, sums_ref, g_ref, be_ref, z_ref):
        mu = sums_ref[0:1, :] * (1.0 / N)
        ms = sums_ref[1:2, :] * (1.0 / N)
        var = ms - mu * mu
        t = g_ref[...] * (y_ref[...] - mu) / jnp.sqrt(var + 1e-5) + be_ref[...]
        z_ref[...] = jnp.where(t > 0, t, jnp.exp(jnp.minimum(t, 0.0)) - 1.0)

    return pl.pallas_call(
        body,
        grid=(GRID,),
        in_specs=[
            pl.BlockSpec((RB, F), lambda i: (i, 0)),
            pl.BlockSpec((2, F), lambda i: (0, 0)),
            pl.BlockSpec((1, F), lambda i: (0, 0)),
            pl.BlockSpec((1, F), lambda i: (0, 0)),
        ],
        out_specs=pl.BlockSpec((RB, F), lambda i: (i, 0)),
        out_shape=jax.ShapeDtypeStruct((N, F), jnp.float32),
    )(y, sums, g.reshape(1, F), be.reshape(1, F))


def _p3(part, xr, b, inv):
    C = xr.shape[1]

    def body(part_ref, xr_ref, b_ref, inv_ref, o_ref):
        t = part_ref[0, :, :C] + part_ref[1, :, :C]
        y = t * inv_ref[...] + xr_ref[...] + b_ref[...]
        m = jnp.max(y, axis=1, keepdims=True)
        s = jnp.sum(jnp.exp(y - m), axis=1, keepdims=True)
        o_ref[...] = (y - m) - jnp.log(s)

    return pl.pallas_call(
        body,
        grid=(GRID,),
        in_specs=[
            pl.BlockSpec((2, RB, Wm), lambda i: (0, i, 0)),
            pl.BlockSpec((RB, C), lambda i: (i, 0)),
            pl.BlockSpec((1, C), lambda i: (0, 0)),
            pl.BlockSpec((RB, 1), lambda i: (i, 0)),
        ],
        out_specs=pl.BlockSpec((RB, C), lambda i: (i, 0)),
        out_shape=jax.ShapeDtypeStruct((N, C), jnp.float32),
    )(part, xr, b.reshape(1, C), inv)


def kernel(x, edge_index, edge_attr, W1, R1, b1, g1, be1,
           W2, R2, b2, g2, be2, W3, R3, b3):
    src = edge_index[0]
    dst = edge_index[1]
    p = edge_attr[:, 0]

    pad = EPT * NW - E
    srcv = jnp.concatenate([src, jnp.zeros((pad,), jnp.int32)]).reshape(NW * NB, BL, K)
    dstv = jnp.concatenate([dst, jnp.full((pad,), DUMMY, jnp.int32)]).reshape(NW * NB, BL, K)
    pv = jnp.concatenate([p, jnp.zeros((pad,), jnp.float32)]).reshape(NW * NB, BL * K)

    wg1 = jnp.concatenate([W1[0], W1[1] - W1[0]], axis=1)
    wg2 = jnp.concatenate([W2[0], W2[1] - W2[0]], axis=1)
    z8 = jnp.zeros((W3.shape[1], 8), jnp.float32)
    z32 = jnp.zeros((W3.shape[1], 32), jnp.float32)
    wg3 = jnp.concatenate([W3[0], z8, W3[1] - W3[0], z8, z32], axis=1)

    hcat1, xr1 = _mm(x, wg1, R1)
    part1, cnt1 = _sc_edge_cnt(hcat1, srcv, dstv, pv)
    part1 = part1.reshape(NC, A_ROWS, Wm)
    cnt1 = cnt1.reshape(NC, A_ROWS, 1)
    y1, sums1, inv = _p1_first(part1, cnt1, xr1, b1)
    z1 = _p2(y1, sums1, g1, be1)

    hcat2, xr2 = _mm(z1, wg2, R2)
    part2 = _sc_edge_128(hcat2, srcv, dstv, pv)[0].reshape(NC, A_ROWS, Wm)
    y2, sums2 = _p1(part2, xr2, b2, inv)
    z2 = _p2(y2, sums2, g2, be2)

    hcat3, xr3 = _mm(z2, wg3, R3)
    part3 = _sc_edge_48(hcat3, srcv, dstv, pv)[0].reshape(NC, A_ROWS, Wm)
    return _p3(part3, xr3, b3, inv)


# speedup vs baseline: 1.2472x; 1.2472x over previous
import functools

import jax
import jax.numpy as jnp
from jax import lax
from jax.experimental import pallas as pl
from jax.experimental.pallas import tpu as pltpu
from jax.experimental.pallas import tpu_sc as plsc

N = 10000
E = 320000
NC = 2
NS = 16
NW = NC * NS
K = 32
BL = 8
NB = 40
CH = BL * NB
EPT = CH * K
STRIPE = 632
A_ROWS = NS * STRIPE
DUMMY = A_ROWS - 1
Wm = 128
CNT_ROWS = 80
GRID = 5
RB = N // GRID


def _make_sc_edge(F, GW, has_cnt):
    mesh = plsc.VectorSubcoreMesh(core_axis_name="c", subcore_axis_name="s",
                                  num_cores=NC, num_subcores=NS)

    out_types = [jax.ShapeDtypeStruct((NC * A_ROWS, Wm), jnp.float32)]
    scratch = [
        pltpu.VMEM_SHARED((A_ROWS, Wm), jnp.float32),
        pltpu.VMEM((BL, K), jnp.int32),
        pltpu.VMEM((BL, K), jnp.int32),
        pltpu.VMEM((BL * K,), jnp.float32),
        pltpu.VMEM((BL, K), jnp.int32),
        pltpu.VMEM((BL, K), jnp.int32),
        pltpu.VMEM((BL * K,), jnp.float32),
        pltpu.VMEM((K, GW), jnp.float32),
        pltpu.VMEM((K, GW), jnp.float32),
        pltpu.VMEM((K, GW), jnp.float32),
        pltpu.VMEM((K, GW), jnp.float32),
        pltpu.VMEM((K, Wm), jnp.float32),
        pltpu.SemaphoreType.DMA,
        pltpu.SemaphoreType.DMA,
        pltpu.SemaphoreType.DMA,
        pltpu.SemaphoreType.DMA,
        pltpu.SemaphoreType.DMA,
        pltpu.SemaphoreType.DMA,
    ]
    if has_cnt:
        out_types.append(jax.ShapeDtypeStruct((NC * CNT_ROWS, 128), jnp.float32))
        scratch += [
            pltpu.VMEM_SHARED((CNT_ROWS, 128), jnp.float32),
            pltpu.VMEM((K,), jnp.int32),
            pltpu.VMEM((K, 128), jnp.float32),
        ]

    @functools.partial(
        pl.kernel,
        out_type=out_types,
        mesh=mesh,
        compiler_params=pltpu.CompilerParams(needs_layout_passes=False),
        scratch_types=scratch,
    )
    def sck(hcat, srcv, dstv, pv, *rest):
        if has_cnt:
            (out, out_cnt, acc, s0_v, d0_v, p0_v, s1_v, d1_v, p1_v,
             r0b, r1b, r2b, r3b, mbuf, semb0, semb1,
             sg0, sg1, sg2, sg3,
             cacc, ccol_v, ohbuf) = rest
        else:
            (out, acc, s0_v, d0_v, p0_v, s1_v, d1_v, p1_v,
             r0b, r1b, r2b, r3b, mbuf, semb0, semb1,
             sg0, sg1, sg2, sg3) = rest
        rows = (r0b, r1b, r2b, r3b)
        sgs = (sg0, sg1, sg2, sg3)
        blks = ((s0_v, d0_v, p0_v, semb0), (s1_v, d1_v, p1_v, semb1))
        cid = lax.axis_index("c")
        sid = lax.axis_index("s")
        wid = cid * NS + sid
        r0 = sid * STRIPE
        base = wid * NB

        def zrow(r, _):
            for j in range(Wm // 16):
                mbuf[r, pl.ds(j * 16, 16)] = jnp.zeros((16,), jnp.float32)
            return 0
        lax.fori_loop(0, K, zrow, 0)

        def zcp(r, _):
            pltpu.sync_copy(mbuf.at[pl.ds(0, 8)],
                            acc.at[pl.ds(r0 + r * 8, 8)])
            return 0
        lax.fori_loop(0, STRIPE // 8, zcp, 0)
        if has_cnt:
            @pl.when(sid < CNT_ROWS // 8)
            def _():
                pltpu.sync_copy(mbuf.at[pl.ds(0, 8)],
                                cacc.at[pl.ds(sid * 8, 8)])

        lanes = lax.iota(jnp.int32, 16)

        def load_block(b, bufs):
            s_r, d_r, p_r, sem = bufs
            pltpu.async_copy(srcv.at[base + b], s_r, sem)
            pltpu.async_copy(dstv.at[base + b], d_r, sem)
            pltpu.async_copy(pv.at[base + b], p_r, sem)

        def wait_block(b, bufs):
            s_r, d_r, p_r, sem = bufs
            pltpu.make_async_copy(srcv.at[base + b], s_r, sem).wait()
            pltpu.make_async_copy(dstv.at[base + b], d_r, sem).wait()
            pltpu.make_async_copy(pv.at[base + b], p_r, sem).wait()

        def startg(s_r, ci, u):
            pltpu.async_copy(hcat.at[s_r.at[ci]], rows[u], sgs[u])

        def waitg(s_r, ci, u):
            pltpu.make_async_copy(hcat.at[s_r.at[ci]], rows[u], sgs[u]).wait()

        def compute(bufs, ci, rows_ref):
            s_r, d_r, p_r, sem = bufs
            if has_cnt:
                def gbody(g, _):
                    dv = d_r[ci, pl.ds(g * 16, 16)]
                    ccol_v[pl.ds(g * 16, 16)] = jnp.bitwise_and(dv, 127)
                    return 0
                lax.fori_loop(0, K // 16, gbody, 0)

            @plsc.parallel_loop(0, K, unroll=4)
            def _(e):
                pe = plsc.load_gather(
                    p_r, [jnp.full((16,), ci * K + e, jnp.int32)])
                for j in range(F // 16):
                    g = rows_ref[e, pl.ds(j * 16, 16)]
                    d = rows_ref[e, pl.ds(F + j * 16, 16)]
                    mbuf[e, pl.ds(j * 16, 16)] = g + pe * d
                if has_cnt:
                    cv = plsc.load_gather(
                        ccol_v, [jnp.full((16,), e, jnp.int32)])
                    for j in range(8):
                        ohbuf[e, pl.ds(j * 16, 16)] = jnp.where(
                            lanes + (j * 16) == cv, 1.0, 0.0)
            pltpu.sync_copy(mbuf, acc.at[d_r.at[ci]], add=True)
            if has_cnt:
                def hbody(g, _):
                    dv = d_r[ci, pl.ds(g * 16, 16)]
                    cidx16 = lax.shift_right_logical(dv, 7)
                    ccol_v[pl.ds(g * 16, 16)] = cidx16
                    return 0
                lax.fori_loop(0, K // 16, hbody, 0)
                pltpu.sync_copy(ohbuf, cacc.at[ccol_v], add=True)

        load_block(0, blks[0])
        wait_block(0, blks[0])
        load_block(1, blks[1])
        plsc.subcore_barrier()
        for u in range(3):
            startg(s0_v, u, u)

        NQ = BL // 4

        def quad(bufs, q, _):
            s_r = bufs[0]
            for u in range(4):
                ci = 4 * q + u
                startg(s_r, ci + 3, (u + 3) % 4)
                waitg(s_r, ci, u)
                compute(bufs, ci, rows[u])
            return 0

        def block_body(b, bufs, nbufs):
            lax.fori_loop(0, NQ - 1, functools.partial(quad, bufs), 0)
            s_r = bufs[0]
            q0 = BL - 4
            startg(s_r, BL - 1, (q0 + 3) % 4)
            waitg(s_r, q0, q0 % 4)
            compute(bufs, q0, rows[q0 % 4])

            @pl.when(b + 1 < NB)
            def _():
                wait_block(b + 1, nbufs)
                startg(nbufs[0], 0, (q0 + 4) % 4)

            waitg(s_r, q0 + 1, (q0 + 1) % 4)
            compute(bufs, q0 + 1, rows[(q0 + 1) % 4])

            @pl.when(b + 1 < NB)
            def _():
                startg(nbufs[0], 1, (q0 + 5) % 4)

            waitg(s_r, q0 + 2, (q0 + 2) % 4)
            compute(bufs, q0 + 2, rows[(q0 + 2) % 4])

            @pl.when(b + 1 < NB)
            def _():
                startg(nbufs[0], 2, (q0 + 6) % 4)

            waitg(s_r, q0 + 3, (q0 + 3) % 4)
            compute(bufs, q0 + 3, rows[(q0 + 3) % 4])

            @pl.when(b + 2 < NB)
            def _():
                load_block(b + 2, bufs)

        def big_body(i, _):
            block_body(2 * i, blks[0], blks[1])
            block_body(2 * i + 1, blks[1], blks[0])
            return 0
        lax.fori_loop(0, NB // 2, big_body, 0)

        plsc.subcore_barrier()
        pltpu.sync_copy(acc.at[pl.ds(r0, STRIPE)],
                        out.at[pl.ds(cid * A_ROWS + r0, STRIPE)])
        if has_cnt:
            @pl.when(sid < CNT_ROWS // 8)
            def _():
                pltpu.sync_copy(
                    cacc.at[pl.ds(sid * 8, 8)],
                    out_cnt.at[pl.ds(cid * CNT_ROWS + sid * 8, 8)])

    return sck


_sc_edge_cnt = _make_sc_edge(128, 256, True)
_sc_edge_128 = _make_sc_edge(128, 256, False)
_sc_edge_48 = _make_sc_edge(48, 128, False)


def _mm(x, wg, r):
    n, din = x.shape
    fg = wg.shape[1]
    fo = r.shape[1]

    def body(x_ref, wg_ref, r_ref, h_ref, xr_ref):
        xb = x_ref[...]
        h_ref[...] = jnp.dot(xb, wg_ref[...], preferred_element_type=jnp.float32)
        xr_ref[...] = jnp.dot(xb, r_ref[...], preferred_element_type=jnp.float32)

    return pl.pallas_call(
        body,
        grid=(GRID,),
        in_specs=[
            pl.BlockSpec((RB, din), lambda i: (i, 0)),
            pl.BlockSpec((din, fg), lambda i: (0, 0)),
            pl.BlockSpec((din, fo), lambda i: (0, 0)),
        ],
        out_specs=[
            pl.BlockSpec((RB, fg), lambda i: (i, 0)),
            pl.BlockSpec((RB, fo), lambda i: (i, 0)),
        ],
        out_shape=[
            jax.ShapeDtypeStruct((n, fg), jnp.float32),
            jax.ShapeDtypeStruct((n, fo), jnp.float32),
        ],
    )(x, wg, r)


def _p1_first(part, cnt, xr, b):
    F = xr.shape[1]

    def body(part_ref, cnt_ref, xr_ref, b_ref, y_ref, sums_ref, inv_ref):
        i = pl.program_id(0)
        t = part_ref[0, :, :F] + part_ref[1, :, :F]
        cntv = cnt_ref[0] + cnt_ref[1]
        inv = 1.0 / jnp.maximum(cntv, 1.0)
        inv_ref[...] = inv
        y = t * inv + xr_ref[...] + b_ref[...]
        y_ref[...] = y

        @pl.when(i == 0)
        def _():
            sums_ref[...] = jnp.zeros_like(sums_ref)
        sums_ref[0, :] += jnp.sum(y, 0)
        sums_ref[1, :] += jnp.sum(y * y, 0)

    return pl.pallas_call(
        body,
        grid=(GRID,),
        in_specs=[
            pl.BlockSpec((2, RB, Wm), lambda i: (0, i, 0)),
            pl.BlockSpec((2, RB, 1), lambda i: (0, i, 0)),
            pl.BlockSpec((RB, F), lambda i: (i, 0)),
            pl.BlockSpec((1, F), lambda i: (0, 0)),
        ],
        out_specs=[
            pl.BlockSpec((RB, F), lambda i: (i, 0)),
            pl.BlockSpec((2, F), lambda i: (0, 0)),
            pl.BlockSpec((RB, 1), lambda i: (i, 0)),
        ],
        out_shape=[
            jax.ShapeDtypeStruct((N, F), jnp.float32),
            jax.ShapeDtypeStruct((2, F), jnp.float32),
            jax.ShapeDtypeStruct((N, 1), jnp.float32),
        ],
    )(part, cnt, xr, b.reshape(1, F))


def _p1(part, xr, b, inv):
    F = xr.shape[1]

    def body(part_ref, xr_ref, b_ref, inv_ref, y_ref, sums_ref):
        i = pl.program_id(0)
        t = part_ref[0, :, :F] + part_ref[1, :, :F]
        y = t * inv_ref[...] + xr_ref[...] + b_ref[...]
        y_ref[...] = y

        @pl.when(i == 0)
        def _():
            sums_ref[...] = jnp.zeros_like(sums_ref)
        sums_ref[0, :] += jnp.sum(y, 0)
        sums_ref[1, :] += jnp.sum(y * y, 0)

    return pl.pallas_call(
        body,
        grid=(GRID,),
        in_specs=[
            pl.BlockSpec((2, RB, Wm), lambda i: (0, i, 0)),
            pl.BlockSpec((RB, F), lambda i: (i, 0)),
            pl.BlockSpec((1, F), lambda i: (0, 0)),
            pl.BlockSpec((RB, 1), lambda i: (i, 0)),
        ],
        out_specs=[
            pl.BlockSpec((RB, F), lambda i: (i, 0)),
            pl.BlockSpec((2, F), lambda i: (0, 0)),
        ],
        out_shape=[
            jax.ShapeDtypeStruct((N, F), jnp.float32),
            jax.ShapeDtypeStruct((2, F), jnp.float32),
        ],
    )(part, xr, b.reshape(1, F), inv)


def _p2(y, sums, g, be):
    F = y.shape[1]

    def body(y_ref, sums_ref, g_ref, be_ref, z_ref):
        mu = sums_ref[0:1, :] * (1.0 / N)
        ms = sums_ref[1:2, :] * (1.0 / N)
        var = ms - mu * mu
        t = g_ref[...] * (y_ref[...] - mu) / jnp.sqrt(var + 1e-5) + be_ref[...]
        z_ref[...] = jnp.where(t > 0, t, jnp.exp(jnp.minimum(t, 0.0)) - 1.0)

    return pl.pallas_call(
        body,
        grid=(GRID,),
        in_specs=[
            pl.BlockSpec((RB, F), lambda i: (i, 0)),
            pl.BlockSpec((2, F), lambda i: (0, 0)),
            pl.BlockSpec((1, F), lambda i: (0, 0)),
            pl.BlockSpec((1, F), lambda i: (0, 0)),
        ],
        out_specs=pl.BlockSpec((RB, F), lambda i: (i, 0)),
        out_shape=jax.ShapeDtypeStruct((N, F), jnp.float32),
    )(y, sums, g.reshape(1, F), be.reshape(1, F))


def _p3(part, xr, b, inv):
    C = xr.shape[1]

    def body(part_ref, xr_ref, b_ref, inv_ref, o_ref):
        t = part_ref[0, :, :C] + part_ref[1, :, :C]
        y = t * inv_ref[...] + xr_ref[...] + b_ref[...]
        m = jnp.max(y, axis=1, keepdims=True)
        s = jnp.sum(jnp.exp(y - m), axis=1, keepdims=True)
        o_ref[...] = (y - m) - jnp.log(s)

    return pl.pallas_call(
        body,
        grid=(GRID,),
        in_specs=[
            pl.BlockSpec((2, RB, Wm), lambda i: (0, i, 0)),
            pl.BlockSpec((RB, C), lambda i: (i, 0)),
            pl.BlockSpec((1, C), lambda i: (0, 0)),
            pl.BlockSpec((RB, 1), lambda i: (i, 0)),
        ],
        out_specs=pl.BlockSpec((RB, C), lambda i: (i, 0)),
        out_shape=jax.ShapeDtypeStruct((N, C), jnp.float32),
    )(part, xr, b.reshape(1, C), inv)


def kernel(x, edge_index, edge_attr, W1, R1, b1, g1, be1,
           W2, R2, b2, g2, be2, W3, R3, b3):
    src = edge_index[0]
    dst = edge_index[1]
    p = edge_attr[:, 0]

    pad = EPT * NW - E
    srcv = jnp.concatenate([src, jnp.zeros((pad,), jnp.int32)]).reshape(NW * NB, BL, K)
    dstv = jnp.concatenate([dst, jnp.full((pad,), DUMMY, jnp.int32)]).reshape(NW * NB, BL, K)
    pv = jnp.concatenate([p, jnp.zeros((pad,), jnp.float32)]).reshape(NW * NB, BL * K)

    wg1 = jnp.concatenate([W1[0], W1[1] - W1[0]], axis=1)
    wg2 = jnp.concatenate([W2[0], W2[1] - W2[0]], axis=1)
    z8 = jnp.zeros((W3.shape[1], 8), jnp.float32)
    z32 = jnp.zeros((W3.shape[1], 32), jnp.float32)
    wg3 = jnp.concatenate([W3[0], z8, W3[1] - W3[0], z8, z32], axis=1)

    hcat1, xr1 = _mm(x, wg1, R1)
    part1, cnt1 = _sc_edge_cnt(hcat1, srcv, dstv, pv)
    part1 = part1.reshape(NC, A_ROWS, Wm)
    cnt1 = cnt1.reshape(NC, CNT_ROWS * 128, 1)
    y1, sums1, inv = _p1_first(part1, cnt1, xr1, b1)
    z1 = _p2(y1, sums1, g1, be1)

    hcat2, xr2 = _mm(z1, wg2, R2)
    part2 = _sc_edge_128(hcat2, srcv, dstv, pv)[0].reshape(NC, A_ROWS, Wm)
    y2, sums2 = _p1(part2, xr2, b2, inv)
    z2 = _p2(y2, sums2, g2, be2)

    hcat3, xr3 = _mm(z2, wg3, R3)
    part3 = _sc_edge_48(hcat3, srcv, dstv, pv)[0].reshape(NC, A_ROWS, Wm)
    return _p3(part3, xr3, b3, inv)


# speedup vs baseline: 1.3355x; 1.0708x over previous
import functools

import jax
import jax.numpy as jnp
from jax import lax
from jax.experimental import pallas as pl
from jax.experimental.pallas import tpu as pltpu
from jax.experimental.pallas import tpu_sc as plsc

N = 10000
E = 320000
NC = 2
NS = 16
NW = NC * NS
K = 32
BL = 8
NB = 40
CH = BL * NB
EPT = CH * K
STRIPE = 632
A_ROWS = NS * STRIPE
DUMMY = A_ROWS - 1
Wm = 128
CNT_ROWS = 80
GRID = 5
RB = N // GRID


def _make_sc_edge(F, GW, has_cnt):
    mesh = plsc.VectorSubcoreMesh(core_axis_name="c", subcore_axis_name="s",
                                  num_cores=NC, num_subcores=NS)

    out_types = [jax.ShapeDtypeStruct((NC * A_ROWS, Wm), jnp.float32)]
    scratch = [
        pltpu.VMEM_SHARED((A_ROWS, Wm), jnp.float32),
        pltpu.VMEM((BL, K), jnp.int32),
        pltpu.VMEM((BL, K), jnp.int32),
        pltpu.VMEM((BL * K,), jnp.float32),
        pltpu.VMEM((BL, K), jnp.int32),
        pltpu.VMEM((BL, K), jnp.int32),
        pltpu.VMEM((BL * K,), jnp.float32),
        pltpu.VMEM((K, GW), jnp.int32),
        pltpu.VMEM((K, GW), jnp.int32),
        pltpu.VMEM((K, GW), jnp.int32),
        pltpu.VMEM((K, GW), jnp.int32),
        pltpu.VMEM((K, Wm), jnp.float32),
        pltpu.SemaphoreType.DMA,
        pltpu.SemaphoreType.DMA,
        pltpu.SemaphoreType.DMA,
        pltpu.SemaphoreType.DMA,
        pltpu.SemaphoreType.DMA,
        pltpu.SemaphoreType.DMA,
    ]
    if has_cnt:
        out_types.append(jax.ShapeDtypeStruct((NC * CNT_ROWS, 128), jnp.float32))
        scratch += [
            pltpu.VMEM_SHARED((CNT_ROWS, 128), jnp.float32),
            pltpu.VMEM((K,), jnp.int32),
            pltpu.VMEM((K, 128), jnp.float32),
        ]

    @functools.partial(
        pl.kernel,
        out_type=out_types,
        mesh=mesh,
        compiler_params=pltpu.CompilerParams(needs_layout_passes=False),
        scratch_types=scratch,
    )
    def sck(hcat, srcv, dstv, pv, *rest):
        if has_cnt:
            (out, out_cnt, acc, s0_v, d0_v, p0_v, s1_v, d1_v, p1_v,
             r0b, r1b, r2b, r3b, mbuf, semb0, semb1,
             sg0, sg1, sg2, sg3,
             cacc, ccol_v, ohbuf) = rest
        else:
            (out, acc, s0_v, d0_v, p0_v, s1_v, d1_v, p1_v,
             r0b, r1b, r2b, r3b, mbuf, semb0, semb1,
             sg0, sg1, sg2, sg3) = rest
        rows = (r0b, r1b, r2b, r3b)
        sgs = (sg0, sg1, sg2, sg3)
        blks = ((s0_v, d0_v, p0_v, semb0), (s1_v, d1_v, p1_v, semb1))
        cid = lax.axis_index("c")
        sid = lax.axis_index("s")
        wid = cid * NS + sid
        r0 = sid * STRIPE
        base = wid * NB

        def zrow(r, _):
            for j in range(Wm // 16):
                mbuf[r, pl.ds(j * 16, 16)] = jnp.zeros((16,), jnp.float32)
            return 0
        lax.fori_loop(0, K, zrow, 0)

        def zcp(r, _):
            pltpu.sync_copy(mbuf.at[pl.ds(0, 8)],
                            acc.at[pl.ds(r0 + r * 8, 8)])
            return 0
        lax.fori_loop(0, STRIPE // 8, zcp, 0)
        if has_cnt:
            @pl.when(sid < CNT_ROWS // 8)
            def _():
                pltpu.sync_copy(mbuf.at[pl.ds(0, 8)],
                                cacc.at[pl.ds(sid * 8, 8)])

        lanes = lax.iota(jnp.int32, 16)

        def load_block(b, bufs):
            s_r, d_r, p_r, sem = bufs
            pltpu.async_copy(srcv.at[base + b], s_r, sem)
            pltpu.async_copy(dstv.at[base + b], d_r, sem)
            pltpu.async_copy(pv.at[base + b], p_r, sem)

        def wait_block(b, bufs):
            s_r, d_r, p_r, sem = bufs
            pltpu.make_async_copy(srcv.at[base + b], s_r, sem).wait()
            pltpu.make_async_copy(dstv.at[base + b], d_r, sem).wait()
            pltpu.make_async_copy(pv.at[base + b], p_r, sem).wait()

        def startg(s_r, ci, u):
            pltpu.async_copy(hcat.at[s_r.at[ci]], rows[u], sgs[u])

        def waitg(s_r, ci, u):
            pltpu.make_async_copy(hcat.at[s_r.at[ci]], rows[u], sgs[u]).wait()

        def compute(bufs, ci, rows_ref):
            s_r, d_r, p_r, sem = bufs
            if has_cnt:
                def gbody(g, _):
                    dv = d_r[ci, pl.ds(g * 16, 16)]
                    ccol_v[pl.ds(g * 16, 16)] = jnp.bitwise_and(dv, 127)
                    return 0
                lax.fori_loop(0, K // 16, gbody, 0)

            @plsc.parallel_loop(0, K, unroll=4)
            def _(e):
                pe = plsc.load_gather(
                    p_r, [jnp.full((16,), ci * K + e, jnp.int32)])
                pe32 = plsc.pack(pe, pe, format=plsc.PackFormat.INTERLEAVED)
                for j in range(F // 32):
                    g = plsc.bitcast(rows_ref[e, pl.ds(j * 16, 16)],
                                     jnp.bfloat16)
                    d = plsc.bitcast(rows_ref[e, pl.ds(F // 2 + j * 16, 16)],
                                     jnp.bfloat16)
                    m32 = g + pe32 * d
                    a, b = plsc.unpack(m32, format=plsc.PackFormat.INTERLEAVED)
                    mbuf[e, pl.ds(j * 32, 16)] = a
                    mbuf[e, pl.ds(j * 32 + 16, 16)] = b
                if has_cnt:
                    cv = plsc.load_gather(
                        ccol_v, [jnp.full((16,), e, jnp.int32)])
                    for j in range(8):
                        ohbuf[e, pl.ds(j * 16, 16)] = jnp.where(
                            lanes + (j * 16) == cv, 1.0, 0.0)
            pltpu.sync_copy(mbuf, acc.at[d_r.at[ci]], add=True)
            if has_cnt:
                def hbody(g, _):
                    dv = d_r[ci, pl.ds(g * 16, 16)]
                    cidx16 = lax.shift_right_logical(dv, 7)
                    ccol_v[pl.ds(g * 16, 16)] = cidx16
                    return 0
                lax.fori_loop(0, K // 16, hbody, 0)
                pltpu.sync_copy(ohbuf, cacc.at[ccol_v], add=True)

        load_block(0, blks[0])
        wait_block(0, blks[0])
        load_block(1, blks[1])
        plsc.subcore_barrier()
        for u in range(3):
            startg(s0_v, u, u)

        NQ = BL // 4

        def quad(bufs, q, _):
            s_r = bufs[0]
            for u in range(4):
                ci = 4 * q + u
                startg(s_r, ci + 3, (u + 3) % 4)
                waitg(s_r, ci, u)
                compute(bufs, ci, rows[u])
            return 0

        def block_body(b, bufs, nbufs):
            lax.fori_loop(0, NQ - 1, functools.partial(quad, bufs), 0)
            s_r = bufs[0]
            q0 = BL - 4
            startg(s_r, BL - 1, (q0 + 3) % 4)
            waitg(s_r, q0, q0 % 4)
            compute(bufs, q0, rows[q0 % 4])

            @pl.when(b + 1 < NB)
            def _():
                wait_block(b + 1, nbufs)
                startg(nbufs[0], 0, (q0 + 4) % 4)

            waitg(s_r, q0 + 1, (q0 + 1) % 4)
            compute(bufs, q0 + 1, rows[(q0 + 1) % 4])

            @pl.when(b + 1 < NB)
            def _():
                startg(nbufs[0], 1, (q0 + 5) % 4)

            waitg(s_r, q0 + 2, (q0 + 2) % 4)
            compute(bufs, q0 + 2, rows[(q0 + 2) % 4])

            @pl.when(b + 1 < NB)
            def _():
                startg(nbufs[0], 2, (q0 + 6) % 4)

            waitg(s_r, q0 + 3, (q0 + 3) % 4)
            compute(bufs, q0 + 3, rows[(q0 + 3) % 4])

            @pl.when(b + 2 < NB)
            def _():
                load_block(b + 2, bufs)

        def big_body(i, _):
            block_body(2 * i, blks[0], blks[1])
            block_body(2 * i + 1, blks[1], blks[0])
            return 0
        lax.fori_loop(0, NB // 2, big_body, 0)

        plsc.subcore_barrier()
        pltpu.sync_copy(acc.at[pl.ds(r0, STRIPE)],
                        out.at[pl.ds(cid * A_ROWS + r0, STRIPE)])
        if has_cnt:
            @pl.when(sid < CNT_ROWS // 8)
            def _():
                pltpu.sync_copy(
                    cacc.at[pl.ds(sid * 8, 8)],
                    out_cnt.at[pl.ds(cid * CNT_ROWS + sid * 8, 8)])

    return sck


_sc_edge_cnt = _make_sc_edge(128, 128, True)
_sc_edge_128 = _make_sc_edge(128, 128, False)
_sc_edge_48 = _make_sc_edge(64, 128, False)


def _perm32(w):
    cols = w.shape[1]
    F = cols // 2
    W = F // 2
    perm = []
    for r in range(2):
        for half in range(2):
            for k in range(W):
                perm.append(r * F + 32 * (k // 16) + half * 16 + (k % 16))
    return w[:, jnp.array(perm, dtype=jnp.int32)]


def _mm(x, wg, r):
    n, din = x.shape
    fg = wg.shape[1]
    fo = r.shape[1]

    W = fg // 4
    out_w = max(fg // 2, 128)

    def body(x_ref, wg_ref, r_ref, h_ref, xr_ref):
        xb = x_ref[...]
        h = jnp.dot(xb, wg_ref[...], preferred_element_type=jnp.float32)
        u = jax.lax.bitcast_convert_type(h, jnp.uint32)
        ub = (u + 0x7FFF + ((u >> 16) & 1)) >> 16
        lo = jnp.concatenate([ub[:, 0:W], ub[:, 2 * W:3 * W]], axis=1)
        hi = jnp.concatenate([ub[:, W:2 * W], ub[:, 3 * W:4 * W]], axis=1)
        packed = lo | (hi << 16)
        if out_w > fg // 2:
            packed = jnp.concatenate(
                [packed,
                 jnp.zeros((packed.shape[0], out_w - fg // 2), jnp.uint32)],
                axis=1)
        h_ref[...] = jax.lax.bitcast_convert_type(packed, jnp.int32)
        xr_ref[...] = jnp.dot(xb, r_ref[...], preferred_element_type=jnp.float32)

    return pl.pallas_call(
        body,
        grid=(GRID,),
        in_specs=[
            pl.BlockSpec((RB, din), lambda i: (i, 0)),
            pl.BlockSpec((din, fg), lambda i: (0, 0)),
            pl.BlockSpec((din, fo), lambda i: (0, 0)),
        ],
        out_specs=[
            pl.BlockSpec((RB, out_w), lambda i: (i, 0)),
            pl.BlockSpec((RB, fo), lambda i: (i, 0)),
        ],
        out_shape=[
            jax.ShapeDtypeStruct((n, out_w), jnp.int32),
            jax.ShapeDtypeStruct((n, fo), jnp.float32),
        ],
    )(x, wg, r)


def _p1_first(part, cnt, xr, b):
    F = xr.shape[1]

    def body(part_ref, cnt_ref, xr_ref, b_ref, y_ref, sums_ref, inv_ref):
        i = pl.program_id(0)
        t = part_ref[0, :, :F] + part_ref[1, :, :F]
        cntv = cnt_ref[0] + cnt_ref[1]
        inv = 1.0 / jnp.maximum(cntv, 1.0)
        inv_ref[...] = inv
        y = t * inv + xr_ref[...] + b_ref[...]
        y_ref[...] = y

        @pl.when(i == 0)
        def _():
            sums_ref[...] = jnp.zeros_like(sums_ref)
        sums_ref[0, :] += jnp.sum(y, 0)
        sums_ref[1, :] += jnp.sum(y * y, 0)

    return pl.pallas_call(
        body,
        grid=(GRID,),
        in_specs=[
            pl.BlockSpec((2, RB, Wm), lambda i: (0, i, 0)),
            pl.BlockSpec((2, RB, 1), lambda i: (0, i, 0)),
            pl.BlockSpec((RB, F), lambda i: (i, 0)),
            pl.BlockSpec((1, F), lambda i: (0, 0)),
        ],
        out_specs=[
            pl.BlockSpec((RB, F), lambda i: (i, 0)),
            pl.BlockSpec((2, F), lambda i: (0, 0)),
            pl.BlockSpec((RB, 1), lambda i: (i, 0)),
        ],
        out_shape=[
            jax.ShapeDtypeStruct((N, F), jnp.float32),
            jax.ShapeDtypeStruct((2, F), jnp.float32),
            jax.ShapeDtypeStruct((N, 1), jnp.float32),
        ],
    )(part, cnt, xr, b.reshape(1, F))


def _p1(part, xr, b, inv):
    F = xr.shape[1]

    def body(part_ref, xr_ref, b_ref, inv_ref, y_ref, sums_ref):
        i = pl.program_id(0)
        t = part_ref[0, :, :F] + part_ref[1, :, :F]
        y = t * inv_ref[...] + xr_ref[...] + b_ref[...]
        y_ref[...] = y

        @pl.when(i == 0)
        def _():
            sums_ref[...] = jnp.zeros_like(sums_ref)
        sums_ref[0, :] += jnp.sum(y, 0)
        sums_ref[1, :] += jnp.sum(y * y, 0)

    return pl.pallas_call(
        body,
        grid=(GRID,),
        in_specs=[
            pl.BlockSpec((2, RB, Wm), lambda i: (0, i, 0)),
            pl.BlockSpec((RB, F), lambda i: (i, 0)),
            pl.BlockSpec((1, F), lambda i: (0, 0)),
            pl.BlockSpec((RB, 1), lambda i: (i, 0)),
        ],
        out_specs=[
            pl.BlockSpec((RB, F), lambda i: (i, 0)),
            pl.BlockSpec((2, F), lambda i: (0, 0)),
        ],
        out_shape=[
            jax.ShapeDtypeStruct((N, F), jnp.float32),
            jax.ShapeDtypeStruct((2, F), jnp.float32),
        ],
    )(part, xr, b.reshape(1, F), inv)


def _p2(y, sums, g, be):
    F = y.shape[1]

    def body(y_ref, sums_ref, g_ref, be_ref, z_ref):
        mu = sums_ref[0:1, :] * (1.0 / N)
        ms = sums_ref[1:2, :] * (1.0 / N)
        var = ms - mu * mu
        t = g_ref[...] * (y_ref[...] - mu) / jnp.sqrt(var + 1e-5) + be_ref[...]
        z_ref[...] = jnp.where(t > 0, t, jnp.exp(jnp.minimum(t, 0.0)) - 1.0)

    return pl.pallas_call(
        body,
        grid=(GRID,),
        in_specs=[
            pl.BlockSpec((RB, F), lambda i: (i, 0)),
            pl.BlockSpec((2, F), lambda i: (0, 0)),
            pl.BlockSpec((1, F), lambda i: (0, 0)),
            pl.BlockSpec((1, F), lambda i: (0, 0)),
        ],
        out_specs=pl.BlockSpec((RB, F), lambda i: (i, 0)),
        out_shape=jax.ShapeDtypeStruct((N, F), jnp.float32),
    )(y, sums, g.reshape(1, F), be.reshape(1, F))


def _p3(part, xr, b, inv):
    C = xr.shape[1]

    def body(part_ref, xr_ref, b_ref, inv_ref, o_ref):
        t = part_ref[0, :, :C] + part_ref[1, :, :C]
        y = t * inv_ref[...] + xr_ref[...] + b_ref[...]
        m = jnp.max(y, axis=1, keepdims=True)
        s = jnp.sum(jnp.exp(y - m), axis=1, keepdims=True)
        o_ref[...] = (y - m) - jnp.log(s)

    return pl.pallas_call(
        body,
        grid=(GRID,),
        in_specs=[
            pl.BlockSpec((2, RB, Wm), lambda i: (0, i, 0)),
            pl.BlockSpec((RB, C), lambda i: (i, 0)),
            pl.BlockSpec((1, C), lambda i: (0, 0)),
            pl.BlockSpec((RB, 1), lambda i: (i, 0)),
        ],
        out_specs=pl.BlockSpec((RB, C), lambda i: (i, 0)),
        out_shape=jax.ShapeDtypeStruct((N, C), jnp.float32),
    )(part, xr, b.reshape(1, C), inv)


def kernel(x, edge_index, edge_attr, W1, R1, b1, g1, be1,
           W2, R2, b2, g2, be2, W3, R3, b3):
    src = edge_index[0]
    dst = edge_index[1]
    p = edge_attr[:, 0]

    pad = EPT * NW - E
    srcv = jnp.concatenate([src, jnp.zeros((pad,), jnp.int32)]).reshape(NW * NB, BL, K)
    dstv = jnp.concatenate([dst, jnp.full((pad,), DUMMY, jnp.int32)]).reshape(NW * NB, BL, K)
    pv = jnp.concatenate([p, jnp.zeros((pad,), jnp.float32)]).reshape(NW * NB, BL * K)

    wg1 = _perm32(jnp.concatenate([W1[0], W1[1] - W1[0]], axis=1))
    wg2 = _perm32(jnp.concatenate([W2[0], W2[1] - W2[0]], axis=1))
    z24 = jnp.zeros((W3.shape[1], 24), jnp.float32)
    wg3 = _perm32(
        jnp.concatenate([W3[0], z24, W3[1] - W3[0], z24], axis=1))

    hcat1, xr1 = _mm(x, wg1, R1)
    part1, cnt1 = _sc_edge_cnt(hcat1, srcv, dstv, pv)
    part1 = part1.reshape(NC, A_ROWS, Wm)
    cnt1 = cnt1.reshape(NC, CNT_ROWS * 128, 1)
    y1, sums1, inv = _p1_first(part1, cnt1, xr1, b1)
    z1 = _p2(y1, sums1, g1, be1)

    hcat2, xr2 = _mm(z1, wg2, R2)
    part2 = _sc_edge_128(hcat2, srcv, dstv, pv)[0].reshape(NC, A_ROWS, Wm)
    y2, sums2 = _p1(part2, xr2, b2, inv)
    z2 = _p2(y2, sums2, g2, be2)

    hcat3, xr3 = _mm(z2, wg3, R3)
    part3 = _sc_edge_48(hcat3, srcv, dstv, pv)[0].reshape(NC, A_ROWS, Wm)
    return _p3(part3, xr3, b3, inv)
